# Initial kernel scaffold; baseline (speedup 1.0000x reference)
import jax
import jax.numpy as jnp
from jax.experimental import pallas as pl


def kernel(buckets, tables, W1, b1, W2, b2):
    raise NotImplementedError("write your pallas kernel here")


# speedup vs baseline: 1.0587x; 1.0587x over previous
import functools

import jax
import jax.numpy as jnp
from jax import lax
from jax.experimental import pallas as pl
from jax.experimental.pallas import tpu as pltpu
from jax.experimental.pallas import tpu_sc as plsc

K = 8
B = 100000
PROJ_DIM = 32
EMB_DIM = 32
HIDDEN = 128
BATCH = 16384

NC = 2
NS = 16
NW = NC * NS
PER_W = BATCH // NW
C = 128
NCHUNK = PER_W // C


def _sc_gather_sum(tab_hbm, idx_hbm, z_hbm, idx_v, rows_v, z_v, sem):
    c = lax.axis_index("c")
    s = lax.axis_index("s")
    wid = s * NC + c

    def chunk_body(ci, carry):
        blk = (wid * NCHUNK + ci) * K
        pltpu.sync_copy(idx_hbm.at[pl.ds(blk, K)], idx_v)

        for k in range(1, K):
            off = k * B

            def add_body(j, _, k=k, off=off):
                sl = pl.ds(j * 16, 16)
                idx_v[k, sl] = idx_v[k, sl] + off
                return 0

            lax.fori_loop(0, C // 16, add_body, 0)

        copies = [
            pltpu.async_copy(tab_hbm.at[idx_v.at[k]], rows_v.at[k], sem)
            for k in range(K)
        ]
        for cop in copies:
            cop.wait()

        def sum_body(r, _):
            for v in range(PROJ_DIM // 16):
                sl = pl.ds(v * 16, 16)
                acc = rows_v[0, r, sl]
                for k in range(1, K):
                    acc = acc + rows_v[k, r, sl]
                z_v[r, sl] = acc
            return 0

        lax.fori_loop(0, C, sum_body, 0)

        row0 = wid * PER_W + ci * C
        pltpu.sync_copy(z_v, z_hbm.at[pl.ds(row0, C)])
        return carry

    lax.fori_loop(0, NCHUNK, chunk_body, 0)


@jax.jit
def _gather_sum(tables_flat, idx):
    mesh = plsc.VectorSubcoreMesh(
        core_axis_name="c", subcore_axis_name="s", num_cores=NC, num_subcores=NS
    )
    return pl.kernel(
        _sc_gather_sum,
        out_type=jax.ShapeDtypeStruct((BATCH, PROJ_DIM), jnp.float32),
        mesh=mesh,
        scratch_types=[
            pltpu.VMEM((K, C), jnp.int32),
            pltpu.VMEM((K, C, PROJ_DIM), jnp.float32),
            pltpu.VMEM((C, PROJ_DIM), jnp.float32),
            pltpu.SemaphoreType.DMA,
        ],
        compiler_params=pltpu.CompilerParams(use_tc_tiling_on_sc=False),
    )(tables_flat, idx)


TB = 2048


def _mlp_body(z_ref, w1_ref, b1_ref, w2_ref, b2_ref, o_ref):
    h = jnp.dot(z_ref[...], w1_ref[...], preferred_element_type=jnp.float32)
    h = jnp.maximum(h + b1_ref[...], 0.0)
    o = jnp.dot(h, w2_ref[...], preferred_element_type=jnp.float32)
    o_ref[...] = o + b2_ref[...]


@jax.jit
def _mlp(z, W1, b1, W2, b2):
    return pl.pallas_call(
        _mlp_body,
        grid=(BATCH // TB,),
        in_specs=[
            pl.BlockSpec((TB, PROJ_DIM), lambda i: (i, 0)),
            pl.BlockSpec((PROJ_DIM, HIDDEN), lambda i: (0, 0)),
            pl.BlockSpec((1, HIDDEN), lambda i: (0, 0)),
            pl.BlockSpec((HIDDEN, EMB_DIM), lambda i: (0, 0)),
            pl.BlockSpec((1, EMB_DIM), lambda i: (0, 0)),
        ],
        out_specs=pl.BlockSpec((TB, EMB_DIM), lambda i: (i, 0)),
        out_shape=jax.ShapeDtypeStruct((BATCH, EMB_DIM), jnp.float32),
    )(z, W1, b1.reshape(1, HIDDEN), W2, b2.reshape(1, EMB_DIM))


def kernel(buckets, tables, W1, b1, W2, b2):
    tables_flat = tables.reshape(K * B, PROJ_DIM)
    idx = (
        buckets.reshape(NW, NCHUNK, C, K)
        .transpose(0, 1, 3, 2)
        .reshape(NW * NCHUNK * K, C)
        .astype(jnp.int32)
    )
    z = _gather_sum(tables_flat, idx)
    return _mlp(z, W1, b1, W2, b2)
